# baseline (speedup 1.0000x reference)
import functools

import jax
import jax.numpy as jnp
from jax import lax
from jax.experimental import pallas as pl
from jax.experimental.pallas import tpu as pltpu
from jax.experimental.pallas import tpu_sc as plsc

N = 50000
E = 800000
D = 64
FW = 128

NC = 2
NS = 16
L = 16

NP = 50176
NPASS = 4
Q = NP // (NC * NPASS)
QT = Q // NS
QP = Q + 8

GCH = 128
CAP = 2304

DGB = 80
DW = 16

E_PER_TILE = E // NS
DCH = 2000
DCH_N = E_PER_TILE // DCH

_mesh = functools.partial(plsc.VectorSubcoreMesh, core_axis_name="c",
                          subcore_axis_name="s", num_cores=NC,
                          num_subcores=NS)


@functools.cache
def _make_deg_kernel():
    return functools.partial(
        pl.kernel,
        out_type=(jax.ShapeDtypeStruct((NP,), jnp.float32),
                  jax.ShapeDtypeStruct((NS * Q,), jnp.float32)),
        mesh=_mesh(),
        compiler_params=pltpu.CompilerParams(needs_layout_passes=False),
        scratch_types=[
            pltpu.VMEM((DCH,), jnp.int32),
            pltpu.VMEM((QP,), jnp.float32),
            pltpu.VMEM((NS * QT,), jnp.float32),
        ],
    )(_deg_body)


def _deg_body(src_hbm, outdeg_hbm, parts_hbm, srcst_v, hist_v, red_v):
    c = lax.axis_index("c")
    s = lax.axis_index("s")
    base = s * E_PER_TILE

    zeros16 = jnp.zeros((L,), jnp.float32)
    ones16 = jnp.ones((L,), jnp.float32)

    @pl.loop(0, NPASS)
    def _(p):
        lo = (NPASS * c + p) * Q

        @pl.loop(0, QP // L)
        def _(i):
            hist_v[pl.ds(i * L, L)] = zeros16

        @pl.loop(0, DCH_N)
        def _(j):
            pltpu.sync_copy(src_hbm.at[pl.ds(base + j * DCH, DCH)], srcst_v)

            @pl.loop(0, DCH // L)
            def _(k):
                v16 = srcst_v[pl.ds(k * L, L)] - lo
                m = v16.astype(jnp.uint32) < jnp.uint32(Q)
                idx16 = jnp.where(m, v16, Q)
                plsc.addupdate_scatter(hist_v, [idx16], ones16)

        pltpu.sync_copy(hist_v.at[pl.ds(0, Q)], parts_hbm.at[pl.ds(s * Q, Q)])
        plsc.subcore_barrier()

        for t in range(NS):
            pltpu.sync_copy(parts_hbm.at[pl.ds(t * Q + s * QT, QT)],
                            red_v.at[pl.ds(t * QT, QT)])

        @pl.loop(0, QT // L)
        def _(i):
            acc = red_v[pl.ds(i * L, L)]
            for t in range(1, NS):
                acc = acc + red_v[pl.ds(t * QT + i * L, L)]
            hist_v[pl.ds(i * L, L)] = acc

        pltpu.sync_copy(hist_v.at[pl.ds(0, QT)],
                        outdeg_hbm.at[pl.ds(lo + s * QT, QT)])
        plsc.subcore_barrier()


def _feat_body(deg_ref, emb_ref, f_ref):
    norm = lax.rsqrt(jnp.maximum(deg_ref[...], 1.0))
    feat = emb_ref[...] * norm
    br = feat.shape[0]
    f_ref[...] = jnp.concatenate(
        [feat, jnp.ones((br, 1), jnp.float32),
         jnp.zeros((br, FW - D - 1), jnp.float32)], axis=1)


_BR = 448
_NB = NP // _BR


def _feat_split(out_deg2d, node_emb):
    return pl.pallas_call(
        _feat_body,
        grid=(_NB,),
        in_specs=[
            pl.BlockSpec((_BR, 1), lambda i: (i, 0)),
            pl.BlockSpec((_BR, D), lambda i: (i, 0)),
        ],
        out_specs=pl.BlockSpec((_BR, FW), lambda i: (i, 0)),
        out_shape=jax.ShapeDtypeStruct((N, FW), jnp.float32),
    )(out_deg2d, node_emb)


@functools.cache
def _make_agg_kernel():
    return functools.partial(
        pl.kernel,
        out_type=jax.ShapeDtypeStruct((NP, FW), jnp.float32),
        mesh=_mesh(),
        compiler_params=pltpu.CompilerParams(needs_layout_passes=False),
        scratch_types=[
            pltpu.VMEM((DCH,), jnp.int32),
            pltpu.VMEM((DCH,), jnp.int32),
            pltpu.VMEM((CAP,), jnp.int32),
            pltpu.VMEM((CAP,), jnp.int32),
            pltpu.VMEM((GCH,), jnp.int32),
            pltpu.VMEM((GCH,), jnp.int32),
            pltpu.VMEM((2, GCH, FW), jnp.float32),
            pltpu.VMEM_SHARED((QP, FW), jnp.float32),
            pltpu.SemaphoreType.DMA,
            pltpu.SemaphoreType.DMA,
        ],
    )(_agg_body)


def _agg_body(feat_hbm, src_hbm, dst_hbm, z2_hbm, agg_hbm,
              srcst_v, dstst_v, csrc_v, cdst_v, dstbuf0_v, dstbuf1_v, rows_v,
              agg_sh, gsem, ssem):
    c = lax.axis_index("c")
    s = lax.axis_index("s")
    base = s * E_PER_TILE
    dstbufs = (dstbuf0_v, dstbuf1_v)

    zero16 = jnp.zeros((L,), jnp.int32)
    dump16 = jnp.full((L,), Q, jnp.int32)

    def drain(nfull):
        @pl.when(nfull > 0)
        def _():
            pltpu.async_copy(feat_hbm.at[csrc_v.at[pl.ds(0, GCH)]],
                             rows_v.at[0], gsem)

        @pl.loop(0, (nfull + 1) // 2)
        def _(h):
            for b in range(2):
                q = h * 2 + b

                @pl.when(q < nfull)
                def _():
                    pltpu.make_async_copy(
                        feat_hbm.at[csrc_v.at[pl.ds(q * GCH, GCH)]],
                        rows_v.at[b], gsem).wait()
                    for i in range(GCH // L):
                        dstbufs[b][pl.ds(i * L, L)] = (
                            cdst_v[pl.ds(q * GCH + i * L, L)])
                    pltpu.async_copy(rows_v.at[b], agg_sh.at[dstbufs[b]],
                                     ssem, add=True)

                    @pl.when(q + 1 < nfull)
                    def _():
                        @pl.when(q >= 1)
                        def _():
                            pltpu.make_async_copy(
                                rows_v.at[1 - b],
                                agg_sh.at[dstbufs[1 - b]], ssem).wait()
                        pltpu.async_copy(
                            feat_hbm.at[csrc_v.at[pl.ds((q + 1) * GCH, GCH)]],
                            rows_v.at[1 - b], gsem)

        @pl.when(nfull >= 1)
        def _():
            pltpu.make_async_copy(rows_v.at[0], agg_sh.at[dstbuf0_v],
                                  ssem).wait()

        @pl.when(nfull >= 2)
        def _():
            pltpu.make_async_copy(rows_v.at[0], agg_sh.at[dstbuf0_v],
                                  ssem).wait()

    @pl.loop(0, NPASS)
    def _(p):
        lo = (NPASS * c + p) * Q

        pltpu.sync_copy(z2_hbm, agg_sh.at[pl.ds(s * QT, QT), :])
        plsc.subcore_barrier()

        def outer(j, off):
            pltpu.sync_copy(src_hbm.at[pl.ds(base + j * DCH, DCH)], srcst_v)
            pltpu.sync_copy(dst_hbm.at[pl.ds(base + j * DCH, DCH)], dstst_v)

            def inner(k, off):
                s16 = srcst_v[pl.ds(k * L, L)]
                d16 = dstst_v[pl.ds(k * L, L)] - lo
                m = d16.astype(jnp.uint32) < jnp.uint32(Q)
                plsc.store_compressed(csrc_v.at[pl.ds(off, L)], s16, mask=m)
                plsc.store_compressed(cdst_v.at[pl.ds(off, L)], d16, mask=m)
                return off + jnp.sum(m.astype(jnp.int32))

            off = lax.fori_loop(0, DCH // L, inner, off)

            nfull = off // GCH
            drain(nfull)

            @pl.when(nfull > 0)
            def _():
                for i in range(GCH // L):
                    csrc_v[pl.ds(i * L, L)] = (
                        csrc_v[pl.ds(nfull * GCH + i * L, L)])
                    cdst_v[pl.ds(i * L, L)] = (
                        cdst_v[pl.ds(nfull * GCH + i * L, L)])

            return off - nfull * GCH

        off = lax.fori_loop(0, DCH_N, outer, 0)

        @pl.when(off > 0)
        def _():
            npad = GCH - off

            @pl.loop(0, GCH // L)
            def _(i):
                m = lax.iota(jnp.int32, L) < (npad - i * L)
                plsc.store_compressed(csrc_v.at[pl.ds(off + i * L, L)],
                                      zero16, mask=m)
                plsc.store_compressed(cdst_v.at[pl.ds(off + i * L, L)],
                                      dump16, mask=m)

            drain(1)

        plsc.subcore_barrier()

        pltpu.sync_copy(agg_sh.at[pl.ds(s * QT, QT), :],
                        agg_hbm.at[pl.ds(lo + s * QT, QT), :])


def _dense_body(agg_ref, w_ref, b_ref, wiht_ref, bih_ref, bhh_ref, out_ref):
    agg = agg_ref[...]
    innorm = lax.rsqrt(jnp.maximum(agg[:, D:D + 1], 1.0))
    a = agg[:, :D] * innorm
    rst = jnp.dot(a, w_ref[...], preferred_element_type=jnp.float32,
                  precision=lax.Precision.HIGHEST) + b_ref[...]
    h = jnp.maximum(rst, 0.0)
    gx = jnp.dot(h, wiht_ref[...], preferred_element_type=jnp.float32,
                 precision=lax.Precision.HIGHEST) + bih_ref[...]
    bhh = bhh_ref[...]
    r = jax.nn.sigmoid(gx[:, :D] + bhh[:, :D])
    z = jax.nn.sigmoid(gx[:, D:2 * D] + bhh[:, D:2 * D])
    nn_ = jnp.tanh(gx[:, 2 * D:] + r * bhh[:, 2 * D:])
    out_ref[...] = (1.0 - z) * nn_


def _dense(agg, W, b, w_ih, b_ih, b_hh):
    wiht = w_ih.T
    full = lambda shape: pl.BlockSpec(shape, lambda i: (0, 0))
    return pl.pallas_call(
        _dense_body,
        grid=(_NB,),
        in_specs=[
            pl.BlockSpec((_BR, FW), lambda i: (i, 0)),
            full((D, D)), full((1, D)),
            full((D, 3 * D)), full((1, 3 * D)), full((1, 3 * D)),
        ],
        out_specs=pl.BlockSpec((_BR, D), lambda i: (i, 0)),
        out_shape=jax.ShapeDtypeStruct((N, D), jnp.float32),
    )(agg, W, b.reshape(1, D), wiht,
      b_ih.reshape(1, 3 * D), b_hh.reshape(1, 3 * D))


def kernel(edge_index, node_emb, W, b, w_ih, w_hh, b_ih, b_hh):
    src = edge_index[0].astype(jnp.int32)
    dst = edge_index[1].astype(jnp.int32)

    z2 = jnp.zeros((QT, FW), jnp.float32)

    table = jnp.zeros((N, FW), jnp.float32).at[:, D].set(1.0)
    zidx = jnp.zeros((E,), jnp.int32)
    out_deg = _make_agg_kernel()(table, zidx, src, z2)[:, D:D + 1]

    feat = _feat_split(out_deg, node_emb)

    agg = _make_agg_kernel()(feat, src, dst, z2)

    return _dense(agg, W, b, w_ih, b_ih, b_hh)


# speedup vs baseline: 14.2658x; 14.2658x over previous
import functools

import jax
import jax.numpy as jnp
from jax import lax
from jax.experimental import pallas as pl
from jax.experimental.pallas import tpu as pltpu
from jax.experimental.pallas import tpu_sc as plsc

N = 50000
E = 800000
D = 64
FW = 128

NC = 2
NS = 16
L = 16

NP = 50176
NPASS = 4
Q = NP // (NC * NPASS)
QT = Q // NS
QP = Q + 8

GCH = 128
CAP = 2304

DGB = 80
DW = 16

E_PER_TILE = E // NS
DCH = 2000
DCH_N = E_PER_TILE // DCH

_mesh = functools.partial(plsc.VectorSubcoreMesh, core_axis_name="c",
                          subcore_axis_name="s", num_cores=NC,
                          num_subcores=NS)


@functools.cache
def _make_deg_kernel():
    return functools.partial(
        pl.kernel,
        out_type=(jax.ShapeDtypeStruct((NP,), jnp.float32),
                  jax.ShapeDtypeStruct((NS * Q,), jnp.float32)),
        mesh=_mesh(),
        compiler_params=pltpu.CompilerParams(needs_layout_passes=False),
        scratch_types=[
            pltpu.VMEM((DCH,), jnp.int32),
            pltpu.VMEM((QP,), jnp.float32),
            pltpu.VMEM((NS * QT,), jnp.float32),
        ],
    )(_deg_body)


def _deg_body(src_hbm, outdeg_hbm, parts_hbm, srcst_v, hist_v, red_v):
    c = lax.axis_index("c")
    s = lax.axis_index("s")
    base = s * E_PER_TILE

    zeros16 = jnp.zeros((L,), jnp.float32)
    ones16 = jnp.ones((L,), jnp.float32)

    @pl.loop(0, NPASS)
    def _(p):
        lo = (NPASS * c + p) * Q

        @pl.loop(0, QP // L)
        def _(i):
            hist_v[pl.ds(i * L, L)] = zeros16

        @pl.loop(0, DCH_N)
        def _(j):
            pltpu.sync_copy(src_hbm.at[pl.ds(base + j * DCH, DCH)], srcst_v)

            @pl.loop(0, DCH // L)
            def _(k):
                v16 = srcst_v[pl.ds(k * L, L)] - lo
                m = v16.astype(jnp.uint32) < jnp.uint32(Q)
                idx16 = jnp.where(m, v16, Q)
                plsc.addupdate_scatter(hist_v, [idx16], ones16)

        pltpu.sync_copy(hist_v.at[pl.ds(0, Q)], parts_hbm.at[pl.ds(s * Q, Q)])
        plsc.subcore_barrier()

        for t in range(NS):
            pltpu.sync_copy(parts_hbm.at[pl.ds(t * Q + s * QT, QT)],
                            red_v.at[pl.ds(t * QT, QT)])

        @pl.loop(0, QT // L)
        def _(i):
            acc = red_v[pl.ds(i * L, L)]
            for t in range(1, NS):
                acc = acc + red_v[pl.ds(t * QT + i * L, L)]
            hist_v[pl.ds(i * L, L)] = acc

        pltpu.sync_copy(hist_v.at[pl.ds(0, QT)],
                        outdeg_hbm.at[pl.ds(lo + s * QT, QT)])
        plsc.subcore_barrier()


def _feat_body(deg_ref, emb_ref, f_ref):
    norm = lax.rsqrt(jnp.maximum(deg_ref[...], 1.0))
    feat = emb_ref[...] * norm
    br = feat.shape[0]
    f_ref[...] = jnp.concatenate(
        [feat, jnp.ones((br, 1), jnp.float32),
         jnp.zeros((br, FW - D - 1), jnp.float32)], axis=1)


_BR = 448
_NB = NP // _BR


def _feat_split(out_deg2d, node_emb):
    return pl.pallas_call(
        _feat_body,
        grid=(_NB,),
        in_specs=[
            pl.BlockSpec((_BR, 1), lambda i: (i, 0)),
            pl.BlockSpec((_BR, D), lambda i: (i, 0)),
        ],
        out_specs=pl.BlockSpec((_BR, FW), lambda i: (i, 0)),
        out_shape=jax.ShapeDtypeStruct((N, FW), jnp.float32),
    )(out_deg2d, node_emb)


@functools.cache
def _make_agg_kernel():
    return functools.partial(
        pl.kernel,
        out_type=jax.ShapeDtypeStruct((NP, FW), jnp.float32),
        mesh=_mesh(),
        compiler_params=pltpu.CompilerParams(needs_layout_passes=False),
        scratch_types=[
            pltpu.VMEM((DCH,), jnp.int32),
            pltpu.VMEM((DCH,), jnp.int32),
            pltpu.VMEM((CAP,), jnp.int32),
            pltpu.VMEM((CAP,), jnp.int32),
            pltpu.VMEM((GCH,), jnp.int32),
            pltpu.VMEM((GCH,), jnp.int32),
            pltpu.VMEM((2, GCH, FW), jnp.float32),
            pltpu.VMEM_SHARED((QP, FW), jnp.float32),
            pltpu.SemaphoreType.DMA,
            pltpu.SemaphoreType.DMA,
        ],
    )(_agg_body)


def _agg_body(feat_hbm, src_hbm, dst_hbm, z2_hbm, agg_hbm,
              srcst_v, dstst_v, csrc_v, cdst_v, dstbuf0_v, dstbuf1_v, rows_v,
              agg_sh, gsem, ssem):
    c = lax.axis_index("c")
    s = lax.axis_index("s")
    base = s * E_PER_TILE
    dstbufs = (dstbuf0_v, dstbuf1_v)

    zero16 = jnp.zeros((L,), jnp.int32)
    dump16 = jnp.full((L,), Q, jnp.int32)

    def drain(nfull):
        @pl.when(nfull > 0)
        def _():
            pltpu.async_copy(feat_hbm.at[csrc_v.at[pl.ds(0, GCH)]],
                             rows_v.at[0], gsem)

        @pl.loop(0, (nfull + 1) // 2)
        def _(h):
            for b in range(2):
                q = h * 2 + b

                @pl.when(q < nfull)
                def _():
                    pltpu.make_async_copy(
                        feat_hbm.at[csrc_v.at[pl.ds(q * GCH, GCH)]],
                        rows_v.at[b], gsem).wait()
                    for i in range(GCH // L):
                        dstbufs[b][pl.ds(i * L, L)] = (
                            cdst_v[pl.ds(q * GCH + i * L, L)])
                    pltpu.async_copy(rows_v.at[b], agg_sh.at[dstbufs[b]],
                                     ssem, add=True)

                    @pl.when(q + 1 < nfull)
                    def _():
                        @pl.when(q >= 1)
                        def _():
                            pltpu.make_async_copy(
                                rows_v.at[1 - b],
                                agg_sh.at[dstbufs[1 - b]], ssem).wait()
                        pltpu.async_copy(
                            feat_hbm.at[csrc_v.at[pl.ds((q + 1) * GCH, GCH)]],
                            rows_v.at[1 - b], gsem)

        @pl.when(nfull >= 1)
        def _():
            pltpu.make_async_copy(rows_v.at[0], agg_sh.at[dstbuf0_v],
                                  ssem).wait()

        @pl.when(nfull >= 2)
        def _():
            pltpu.make_async_copy(rows_v.at[0], agg_sh.at[dstbuf0_v],
                                  ssem).wait()

    @pl.loop(0, NPASS)
    def _(p):
        lo = (NPASS * c + p) * Q

        pltpu.sync_copy(z2_hbm, agg_sh.at[pl.ds(s * QT, QT), :])
        plsc.subcore_barrier()

        def outer(j, off):
            pltpu.sync_copy(src_hbm.at[pl.ds(base + j * DCH, DCH)], srcst_v)
            pltpu.sync_copy(dst_hbm.at[pl.ds(base + j * DCH, DCH)], dstst_v)

            def inner(k, off):
                s16 = srcst_v[pl.ds(k * L, L)]
                d16 = dstst_v[pl.ds(k * L, L)] - lo
                m = d16.astype(jnp.uint32) < jnp.uint32(Q)
                plsc.store_compressed(csrc_v.at[pl.ds(off, L)], s16, mask=m)
                plsc.store_compressed(cdst_v.at[pl.ds(off, L)], d16, mask=m)
                return off + jnp.sum(m.astype(jnp.int32))

            off = lax.fori_loop(0, DCH // L, inner, off)

            nfull = off // GCH
            drain(nfull)

            @pl.when(nfull > 0)
            def _():
                for i in range(GCH // L):
                    csrc_v[pl.ds(i * L, L)] = (
                        csrc_v[pl.ds(nfull * GCH + i * L, L)])
                    cdst_v[pl.ds(i * L, L)] = (
                        cdst_v[pl.ds(nfull * GCH + i * L, L)])

            return off - nfull * GCH

        off = lax.fori_loop(0, DCH_N, outer, 0)

        @pl.when(off > 0)
        def _():
            npad = GCH - off

            @pl.loop(0, GCH // L)
            def _(i):
                m = lax.iota(jnp.int32, L) < (npad - i * L)
                plsc.store_compressed(csrc_v.at[pl.ds(off + i * L, L)],
                                      zero16, mask=m)
                plsc.store_compressed(cdst_v.at[pl.ds(off + i * L, L)],
                                      dump16, mask=m)

            drain(1)

        plsc.subcore_barrier()

        pltpu.sync_copy(agg_sh.at[pl.ds(s * QT, QT), :],
                        agg_hbm.at[pl.ds(lo + s * QT, QT), :])


def _dense_body(agg_ref, w_ref, b_ref, wiht_ref, bih_ref, bhh_ref, out_ref):
    agg = agg_ref[...]
    innorm = lax.rsqrt(jnp.maximum(agg[:, D:D + 1], 1.0))
    a = agg[:, :D] * innorm
    rst = jnp.dot(a, w_ref[...], preferred_element_type=jnp.float32,
                  precision=lax.Precision.HIGHEST) + b_ref[...]
    h = jnp.maximum(rst, 0.0)
    gx = jnp.dot(h, wiht_ref[...], preferred_element_type=jnp.float32,
                 precision=lax.Precision.HIGHEST) + bih_ref[...]
    bhh = bhh_ref[...]
    r = jax.nn.sigmoid(gx[:, :D] + bhh[:, :D])
    z = jax.nn.sigmoid(gx[:, D:2 * D] + bhh[:, D:2 * D])
    nn_ = jnp.tanh(gx[:, 2 * D:] + r * bhh[:, 2 * D:])
    out_ref[...] = (1.0 - z) * nn_


def _dense(agg, W, b, w_ih, b_ih, b_hh):
    wiht = w_ih.T
    full = lambda shape: pl.BlockSpec(shape, lambda i: (0, 0))
    return pl.pallas_call(
        _dense_body,
        grid=(_NB,),
        in_specs=[
            pl.BlockSpec((_BR, FW), lambda i: (i, 0)),
            full((D, D)), full((1, D)),
            full((D, 3 * D)), full((1, 3 * D)), full((1, 3 * D)),
        ],
        out_specs=pl.BlockSpec((_BR, D), lambda i: (i, 0)),
        out_shape=jax.ShapeDtypeStruct((N, D), jnp.float32),
    )(agg, W, b.reshape(1, D), wiht,
      b_ih.reshape(1, 3 * D), b_hh.reshape(1, 3 * D))


def kernel(edge_index, node_emb, W, b, w_ih, w_hh, b_ih, b_hh):
    src = edge_index[0].astype(jnp.int32)
    dst = edge_index[1].astype(jnp.int32)

    z2 = jnp.zeros((QT, FW), jnp.float32)

    ones_deg = jnp.ones((NP, 1), jnp.float32)
    embp = _feat_split(ones_deg, node_emb)
    out_deg = _make_agg_kernel()(embp, src, src, z2)[:, D:D + 1]

    feat = _feat_split(out_deg, node_emb)

    agg = _make_agg_kernel()(feat, src, dst, z2)

    return _dense(agg, W, b, w_ih, b_ih, b_hh)


# speedup vs baseline: 15.3437x; 1.0756x over previous
import functools

import jax
import jax.numpy as jnp
from jax import lax
from jax.experimental import pallas as pl
from jax.experimental.pallas import tpu as pltpu
from jax.experimental.pallas import tpu_sc as plsc

N = 50000
E = 800000
D = 64
FW = 128

NC = 2
NS = 16
L = 16

NP = 50176
NPASS = 4
Q = NP // (NC * NPASS)
QT = Q // NS
QP = Q + 8

GCH = 128
CAP = 2304

EP = 802816
E_PER_TILE = EP // NS
DCH = 1568
DCH_N = E_PER_TILE // DCH
EPAD_IDX = 60000

_mesh = functools.partial(plsc.VectorSubcoreMesh, core_axis_name="c",
                          subcore_axis_name="s", num_cores=NC,
                          num_subcores=NS)


def _feat_body(deg_ref, emb_ref, f_ref):
    norm = lax.rsqrt(jnp.maximum(deg_ref[...], 1.0))
    feat = emb_ref[...] * norm
    br = feat.shape[0]
    f_ref[...] = jnp.concatenate(
        [feat, jnp.ones((br, 1), jnp.float32),
         jnp.zeros((br, FW - D - 1), jnp.float32)], axis=1)


_BR = 448
_NB = NP // _BR


def _feat_split(out_deg2d, node_emb):
    return pl.pallas_call(
        _feat_body,
        grid=(_NB,),
        in_specs=[
            pl.BlockSpec((_BR, 1), lambda i: (i, 0)),
            pl.BlockSpec((_BR, D), lambda i: (i, 0)),
        ],
        out_specs=pl.BlockSpec((_BR, FW), lambda i: (i, 0)),
        out_shape=jax.ShapeDtypeStruct((N, FW), jnp.float32),
    )(out_deg2d, node_emb)


@functools.cache
def _make_agg_kernel():
    return functools.partial(
        pl.kernel,
        out_type=jax.ShapeDtypeStruct((NP, FW), jnp.float32),
        mesh=_mesh(),
        compiler_params=pltpu.CompilerParams(needs_layout_passes=False),
        scratch_types=[
            pltpu.VMEM((DCH,), jnp.int32),
            pltpu.VMEM((DCH,), jnp.int32),
            pltpu.VMEM((DCH,), jnp.int32),
            pltpu.VMEM((DCH,), jnp.int32),
            pltpu.VMEM((CAP,), jnp.int32),
            pltpu.VMEM((CAP,), jnp.int32),
            pltpu.VMEM((GCH,), jnp.int32),
            pltpu.VMEM((GCH,), jnp.int32),
            pltpu.VMEM((2, GCH, FW), jnp.float32),
            pltpu.VMEM_SHARED((QP, FW), jnp.float32),
            pltpu.SemaphoreType.DMA,
            pltpu.SemaphoreType.DMA,
            pltpu.SemaphoreType.DMA,
            pltpu.SemaphoreType.DMA,
        ],
    )(_agg_body)


def _agg_body(feat_hbm, src_hbm, dst_hbm, z2_hbm, agg_hbm,
              srcst0_v, srcst1_v, dstst0_v, dstst1_v, csrc_v, cdst_v,
              dstbuf0_v, dstbuf1_v, rows_v, agg_sh, gsem, ssem, st0, st1):
    c = lax.axis_index("c")
    s = lax.axis_index("s")
    base = s * E_PER_TILE
    dstbufs = (dstbuf0_v, dstbuf1_v)
    srcsts = (srcst0_v, srcst1_v)
    dststs = (dstst0_v, dstst1_v)
    stsems = (st0, st1)

    zero16 = jnp.zeros((L,), jnp.int32)
    dump16 = jnp.full((L,), Q, jnp.int32)

    def drain(nfull):
        @pl.when(nfull > 0)
        def _():
            pltpu.async_copy(feat_hbm.at[csrc_v.at[pl.ds(0, GCH)]],
                             rows_v.at[0], gsem)

        @pl.loop(0, (nfull + 1) // 2)
        def _(h):
            for b in range(2):
                q = h * 2 + b

                @pl.when(q < nfull)
                def _():
                    pltpu.make_async_copy(
                        feat_hbm.at[csrc_v.at[pl.ds(q * GCH, GCH)]],
                        rows_v.at[b], gsem).wait()
                    for i in range(GCH // L):
                        dstbufs[b][pl.ds(i * L, L)] = (
                            cdst_v[pl.ds(q * GCH + i * L, L)])
                    pltpu.async_copy(rows_v.at[b], agg_sh.at[dstbufs[b]],
                                     ssem, add=True)

                    @pl.when(q + 1 < nfull)
                    def _():
                        @pl.when(q >= 1)
                        def _():
                            pltpu.make_async_copy(
                                rows_v.at[1 - b],
                                agg_sh.at[dstbufs[1 - b]], ssem).wait()
                        pltpu.async_copy(
                            feat_hbm.at[csrc_v.at[pl.ds((q + 1) * GCH, GCH)]],
                            rows_v.at[1 - b], gsem)

        @pl.when(nfull >= 1)
        def _():
            pltpu.make_async_copy(rows_v.at[0], agg_sh.at[dstbuf0_v],
                                  ssem).wait()

        @pl.when(nfull >= 2)
        def _():
            pltpu.make_async_copy(rows_v.at[0], agg_sh.at[dstbuf0_v],
                                  ssem).wait()

    @pl.loop(0, NPASS)
    def _(p):
        lo = (NPASS * c + p) * Q

        pltpu.sync_copy(z2_hbm, agg_sh.at[pl.ds(s * QT, QT), :])
        plsc.subcore_barrier()

        for b in range(2):
            pltpu.async_copy(src_hbm.at[pl.ds(base + b * DCH, DCH)],
                             srcsts[b], stsems[b])
            pltpu.async_copy(dst_hbm.at[pl.ds(base + b * DCH, DCH)],
                             dststs[b], stsems[b])

        def outer(h, off):
            for b in range(2):
                j = 2 * h + b
                pltpu.make_async_copy(src_hbm.at[pl.ds(0, DCH)], srcsts[b],
                                      stsems[b]).wait()
                pltpu.make_async_copy(dst_hbm.at[pl.ds(0, DCH)], dststs[b],
                                      stsems[b]).wait()

                def inner(k, off):
                    s16 = srcsts[b][pl.ds(k * L, L)]
                    d16 = dststs[b][pl.ds(k * L, L)] - lo
                    m = d16.astype(jnp.uint32) < jnp.uint32(Q)
                    plsc.store_compressed(csrc_v.at[pl.ds(off, L)], s16,
                                          mask=m)
                    plsc.store_compressed(cdst_v.at[pl.ds(off, L)], d16,
                                          mask=m)
                    return off + plsc.all_reduce_population_count(m)[0]

                off = lax.fori_loop(0, DCH // L, inner, off)

                @pl.when(j + 2 < DCH_N)
                def _():
                    nb = base + (j + 2) * DCH
                    pltpu.async_copy(src_hbm.at[pl.ds(nb, DCH)], srcsts[b],
                                     stsems[b])
                    pltpu.async_copy(dst_hbm.at[pl.ds(nb, DCH)], dststs[b],
                                     stsems[b])

                nfull = off // GCH
                drain(nfull)

                @pl.when(nfull > 0)
                def _():
                    for i in range(GCH // L):
                        csrc_v[pl.ds(i * L, L)] = (
                            csrc_v[pl.ds(nfull * GCH + i * L, L)])
                        cdst_v[pl.ds(i * L, L)] = (
                            cdst_v[pl.ds(nfull * GCH + i * L, L)])

                off = off - nfull * GCH
            return off

        off = lax.fori_loop(0, DCH_N // 2, outer, 0)

        @pl.when(off > 0)
        def _():
            npad = GCH - off

            @pl.loop(0, GCH // L)
            def _(i):
                m = lax.iota(jnp.int32, L) < (npad - i * L)
                plsc.store_compressed(csrc_v.at[pl.ds(off + i * L, L)],
                                      zero16, mask=m)
                plsc.store_compressed(cdst_v.at[pl.ds(off + i * L, L)],
                                      dump16, mask=m)

            drain(1)

        plsc.subcore_barrier()

        pltpu.sync_copy(agg_sh.at[pl.ds(s * QT, QT), :],
                        agg_hbm.at[pl.ds(lo + s * QT, QT), :])


def _dense_body(agg_ref, w_ref, b_ref, wiht_ref, bih_ref, bhh_ref, out_ref):
    agg = agg_ref[...]
    innorm = lax.rsqrt(jnp.maximum(agg[:, D:D + 1], 1.0))
    a = agg[:, :D] * innorm
    rst = jnp.dot(a, w_ref[...], preferred_element_type=jnp.float32,
                  precision=lax.Precision.HIGHEST) + b_ref[...]
    h = jnp.maximum(rst, 0.0)
    gx = jnp.dot(h, wiht_ref[...], preferred_element_type=jnp.float32,
                 precision=lax.Precision.HIGHEST) + bih_ref[...]
    bhh = bhh_ref[...]
    r = jax.nn.sigmoid(gx[:, :D] + bhh[:, :D])
    z = jax.nn.sigmoid(gx[:, D:2 * D] + bhh[:, D:2 * D])
    nn_ = jnp.tanh(gx[:, 2 * D:] + r * bhh[:, 2 * D:])
    out_ref[...] = (1.0 - z) * nn_


def _dense(agg, W, b, w_ih, b_ih, b_hh):
    wiht = w_ih.T
    full = lambda shape: pl.BlockSpec(shape, lambda i: (0, 0))
    return pl.pallas_call(
        _dense_body,
        grid=(_NB,),
        in_specs=[
            pl.BlockSpec((_BR, FW), lambda i: (i, 0)),
            full((D, D)), full((1, D)),
            full((D, 3 * D)), full((1, 3 * D)), full((1, 3 * D)),
        ],
        out_specs=pl.BlockSpec((_BR, D), lambda i: (i, 0)),
        out_shape=jax.ShapeDtypeStruct((N, D), jnp.float32),
    )(agg, W, b.reshape(1, D), wiht,
      b_ih.reshape(1, 3 * D), b_hh.reshape(1, 3 * D))


def kernel(edge_index, node_emb, W, b, w_ih, w_hh, b_ih, b_hh):
    pad = jnp.full((EP - E,), EPAD_IDX, jnp.int32)
    src = jnp.concatenate([edge_index[0].astype(jnp.int32), pad])
    dst = jnp.concatenate([edge_index[1].astype(jnp.int32), pad])

    z2 = jnp.zeros((QT, FW), jnp.float32)

    ones_deg = jnp.ones((NP, 1), jnp.float32)
    embp = _feat_split(ones_deg, node_emb)
    out_deg = _make_agg_kernel()(embp, src, src, z2)[:, D:D + 1]

    feat = _feat_split(out_deg, node_emb)

    agg = _make_agg_kernel()(feat, src, dst, z2)

    return _dense(agg, W, b, w_ih, b_ih, b_hh)


# speedup vs baseline: 15.5975x; 1.0165x over previous
import functools

import jax
import jax.numpy as jnp
from jax import lax
from jax.experimental import pallas as pl
from jax.experimental.pallas import tpu as pltpu
from jax.experimental.pallas import tpu_sc as plsc

N = 50000
E = 800000
D = 64
FW = 128

NC = 2
NS = 16
L = 16

NP = 50176
NPASS = 4
Q = NP // (NC * NPASS)
QT = Q // NS
QP = Q + 8

GCH = 128
CAP = 2304

EP = 802816
E_PER_TILE = EP // NS
DCH = 1568
DCH_N = E_PER_TILE // DCH
EPAD_IDX = 60000

_mesh = functools.partial(plsc.VectorSubcoreMesh, core_axis_name="c",
                          subcore_axis_name="s", num_cores=NC,
                          num_subcores=NS)


def _feat_body(deg_ref, emb_ref, f_ref):
    norm = lax.rsqrt(jnp.maximum(deg_ref[...], 1.0))
    feat = emb_ref[...] * norm
    br = feat.shape[0]
    f_ref[...] = jnp.concatenate(
        [feat, jnp.ones((br, 1), jnp.float32),
         jnp.zeros((br, FW - D - 1), jnp.float32)], axis=1)


_BR = 448
_NB = NP // _BR


def _feat_split(out_deg2d, node_emb):
    return pl.pallas_call(
        _feat_body,
        grid=(_NB,),
        in_specs=[
            pl.BlockSpec((_BR, 1), lambda i: (i, 0)),
            pl.BlockSpec((_BR, D), lambda i: (i, 0)),
        ],
        out_specs=pl.BlockSpec((_BR, FW), lambda i: (i, 0)),
        out_shape=jax.ShapeDtypeStruct((N, FW), jnp.float32),
    )(out_deg2d, node_emb)


@functools.cache
def _make_agg_kernel():
    return functools.partial(
        pl.kernel,
        out_type=jax.ShapeDtypeStruct((NP, FW), jnp.float32),
        mesh=_mesh(),
        compiler_params=pltpu.CompilerParams(needs_layout_passes=False),
        scratch_types=[
            pltpu.VMEM((DCH,), jnp.int32),
            pltpu.VMEM((DCH,), jnp.int32),
            pltpu.VMEM((DCH,), jnp.int32),
            pltpu.VMEM((DCH,), jnp.int32),
            pltpu.VMEM((CAP,), jnp.int32),
            pltpu.VMEM((CAP,), jnp.int32),
            pltpu.VMEM((GCH,), jnp.int32),
            pltpu.VMEM((GCH,), jnp.int32),
            pltpu.VMEM((2, GCH, FW), jnp.float32),
            pltpu.VMEM_SHARED((QP, FW), jnp.float32),
            pltpu.SemaphoreType.DMA,
            pltpu.SemaphoreType.DMA,
            pltpu.SemaphoreType.DMA,
            pltpu.SemaphoreType.DMA,
        ],
    )(_agg_body)


def _agg_body(feat_hbm, src_hbm, dst_hbm, z2_hbm, agg_hbm,
              srcst0_v, srcst1_v, dstst0_v, dstst1_v, csrc_v, cdst_v,
              dstbuf0_v, dstbuf1_v, rows_v, agg_sh, gsem, ssem, st0, st1):
    c = lax.axis_index("c")
    s = lax.axis_index("s")
    base = s * E_PER_TILE
    dstbufs = (dstbuf0_v, dstbuf1_v)
    srcsts = (srcst0_v, srcst1_v)
    dststs = (dstst0_v, dstst1_v)
    stsems = (st0, st1)

    zero16 = jnp.zeros((L,), jnp.int32)
    dump16 = jnp.full((L,), Q, jnp.int32)

    def drain(nfull):
        @pl.when(nfull > 0)
        def _():
            pltpu.async_copy(feat_hbm.at[csrc_v.at[pl.ds(0, GCH)]],
                             rows_v.at[0], gsem)

        @pl.loop(0, (nfull + 1) // 2)
        def _(h):
            for b in range(2):
                q = h * 2 + b

                @pl.when(q < nfull)
                def _():
                    pltpu.make_async_copy(
                        feat_hbm.at[csrc_v.at[pl.ds(q * GCH, GCH)]],
                        rows_v.at[b], gsem).wait()
                    for i in range(GCH // L):
                        dstbufs[b][pl.ds(i * L, L)] = (
                            cdst_v[pl.ds(q * GCH + i * L, L)])
                    pltpu.async_copy(rows_v.at[b], agg_sh.at[dstbufs[b]],
                                     ssem, add=True)

                    @pl.when(q + 1 < nfull)
                    def _():
                        @pl.when(q >= 1)
                        def _():
                            pltpu.make_async_copy(
                                rows_v.at[1 - b],
                                agg_sh.at[dstbufs[1 - b]], ssem).wait()
                        pltpu.async_copy(
                            feat_hbm.at[csrc_v.at[pl.ds((q + 1) * GCH, GCH)]],
                            rows_v.at[1 - b], gsem)

        @pl.when(nfull >= 1)
        def _():
            pltpu.make_async_copy(rows_v.at[0], agg_sh.at[dstbuf0_v],
                                  ssem).wait()

        @pl.when(nfull >= 2)
        def _():
            pltpu.make_async_copy(rows_v.at[0], agg_sh.at[dstbuf0_v],
                                  ssem).wait()

    @pl.loop(0, NPASS)
    def _(p):
        lo = (NPASS * c + p) * Q

        pltpu.sync_copy(z2_hbm, agg_sh.at[pl.ds(s * QT, QT), :])
        plsc.subcore_barrier()

        for b in range(2):
            pltpu.async_copy(src_hbm.at[pl.ds(base + b * DCH, DCH)],
                             srcsts[b], stsems[b])
            pltpu.async_copy(dst_hbm.at[pl.ds(base + b * DCH, DCH)],
                             dststs[b], stsems[b])

        def outer(h, off):
            for b in range(2):
                j = 2 * h + b
                pltpu.make_async_copy(src_hbm.at[pl.ds(0, DCH)], srcsts[b],
                                      stsems[b]).wait()
                pltpu.make_async_copy(dst_hbm.at[pl.ds(0, DCH)], dststs[b],
                                      stsems[b]).wait()

                def inner(k, off):
                    s16 = srcsts[b][pl.ds(k * L, L)]
                    d16 = dststs[b][pl.ds(k * L, L)] - lo
                    m = d16.astype(jnp.uint32) < jnp.uint32(Q)
                    plsc.store_compressed(csrc_v.at[pl.ds(off, L)], s16,
                                          mask=m)
                    plsc.store_compressed(cdst_v.at[pl.ds(off, L)], d16,
                                          mask=m)
                    return off + plsc.all_reduce_population_count(m)[0]

                off = lax.fori_loop(0, DCH // L, inner, off, unroll=2)

                @pl.when(j + 2 < DCH_N)
                def _():
                    nb = base + (j + 2) * DCH
                    pltpu.async_copy(src_hbm.at[pl.ds(nb, DCH)], srcsts[b],
                                     stsems[b])
                    pltpu.async_copy(dst_hbm.at[pl.ds(nb, DCH)], dststs[b],
                                     stsems[b])

                nfull = off // GCH
                drain(nfull)

                @pl.when(nfull > 0)
                def _():
                    for i in range(GCH // L):
                        csrc_v[pl.ds(i * L, L)] = (
                            csrc_v[pl.ds(nfull * GCH + i * L, L)])
                        cdst_v[pl.ds(i * L, L)] = (
                            cdst_v[pl.ds(nfull * GCH + i * L, L)])

                off = off - nfull * GCH
            return off

        off = lax.fori_loop(0, DCH_N // 2, outer, 0)

        @pl.when(off > 0)
        def _():
            npad = GCH - off

            @pl.loop(0, GCH // L)
            def _(i):
                m = lax.iota(jnp.int32, L) < (npad - i * L)
                plsc.store_compressed(csrc_v.at[pl.ds(off + i * L, L)],
                                      zero16, mask=m)
                plsc.store_compressed(cdst_v.at[pl.ds(off + i * L, L)],
                                      dump16, mask=m)

            drain(1)

        plsc.subcore_barrier()

        pltpu.sync_copy(agg_sh.at[pl.ds(s * QT, QT), :],
                        agg_hbm.at[pl.ds(lo + s * QT, QT), :])


def _dense_body(agg_ref, w_ref, b_ref, wiht_ref, bih_ref, bhh_ref, out_ref):
    agg = agg_ref[...]
    innorm = lax.rsqrt(jnp.maximum(agg[:, D:D + 1], 1.0))
    a = agg[:, :D] * innorm
    rst = jnp.dot(a, w_ref[...], preferred_element_type=jnp.float32,
                  precision=lax.Precision.HIGHEST) + b_ref[...]
    h = jnp.maximum(rst, 0.0)
    gx = jnp.dot(h, wiht_ref[...], preferred_element_type=jnp.float32,
                 precision=lax.Precision.HIGHEST) + bih_ref[...]
    bhh = bhh_ref[...]
    r = jax.nn.sigmoid(gx[:, :D] + bhh[:, :D])
    z = jax.nn.sigmoid(gx[:, D:2 * D] + bhh[:, D:2 * D])
    nn_ = jnp.tanh(gx[:, 2 * D:] + r * bhh[:, 2 * D:])
    out_ref[...] = (1.0 - z) * nn_


def _dense(agg, W, b, w_ih, b_ih, b_hh):
    wiht = w_ih.T
    full = lambda shape: pl.BlockSpec(shape, lambda i: (0, 0))
    return pl.pallas_call(
        _dense_body,
        grid=(_NB,),
        in_specs=[
            pl.BlockSpec((_BR, FW), lambda i: (i, 0)),
            full((D, D)), full((1, D)),
            full((D, 3 * D)), full((1, 3 * D)), full((1, 3 * D)),
        ],
        out_specs=pl.BlockSpec((_BR, D), lambda i: (i, 0)),
        out_shape=jax.ShapeDtypeStruct((N, D), jnp.float32),
    )(agg, W, b.reshape(1, D), wiht,
      b_ih.reshape(1, 3 * D), b_hh.reshape(1, 3 * D))


def kernel(edge_index, node_emb, W, b, w_ih, w_hh, b_ih, b_hh):
    pad = jnp.full((EP - E,), EPAD_IDX, jnp.int32)
    src = jnp.concatenate([edge_index[0].astype(jnp.int32), pad])
    dst = jnp.concatenate([edge_index[1].astype(jnp.int32), pad])

    z2 = jnp.zeros((QT, FW), jnp.float32)

    ones_deg = jnp.ones((NP, 1), jnp.float32)
    embp = _feat_split(ones_deg, node_emb)
    out_deg = _make_agg_kernel()(embp, src, src, z2)[:, D:D + 1]

    feat = _feat_split(out_deg, node_emb)

    agg = _make_agg_kernel()(feat, src, dst, z2)

    return _dense(agg, W, b, w_ih, b_ih, b_hh)
